# baseline (speedup 1.0000x reference)
import functools

import jax
import jax.numpy as jnp
from jax import lax
from jax.experimental import pallas as pl
from jax.experimental.pallas import tpu as pltpu
from jax.experimental.pallas import tpu_sc as plsc

_S = 7.0
_N = 90
_LC = 5.0
_LN = 0.5
_BATCH = 4096
_M = _BATCH * 49
_NW = 32
_CELLS_W = _M // _NW
_CHUNK = 224
_NCH = _CELLS_W // _CHUNK
_CW = _CHUNK * _N
_GRP = _CHUNK // 16

_mesh = plsc.VectorSubcoreMesh(core_axis_name="c", subcore_axis_name="s")


_TOFF = _M * _N


@functools.partial(
    pl.kernel,
    out_type=jax.ShapeDtypeStruct((_NW, 16), jnp.float32),
    mesh=_mesh,
    scratch_types=[
        pltpu.VMEM((_CW,), jnp.float32),
        pltpu.VMEM((_CW,), jnp.float32),
        pltpu.VMEM((_CW,), jnp.float32),
        pltpu.VMEM((_CW,), jnp.float32),
        pltpu.VMEM((16,), jnp.float32),
        pltpu.SemaphoreType.DMA,
        pltpu.SemaphoreType.DMA,
    ],
    compiler_params=pltpu.CompilerParams(needs_layout_passes=False),
)
def _sc_loss(x_hbm, out_hbm, pv0, tv0, pv1, tv1, acc_v, sem0, sem1):
    wid = lax.axis_index("s") * 2 + lax.axis_index("c")
    iota = lax.broadcasted_iota(jnp.int32, (16,), 0)
    idx90 = iota * _N
    zero = jnp.zeros((16,), jnp.float32)

    def start_chunk(k, pv, tv, sem):
        start = (wid * _CELLS_W + k * _CHUNK) * _N
        pltpu.async_copy(x_hbm.at[pl.ds(start, _CW)], pv, sem)
        pltpu.async_copy(x_hbm.at[pl.ds(start + _TOFF, _CW)], tv, sem)

    def wait_chunk(pv, tv, sem):
        pltpu.make_async_copy(x_hbm.at[pl.ds(0, _CW)], pv, sem).wait()
        pltpu.make_async_copy(x_hbm.at[pl.ds(0, _CW)], tv, sem).wait()

    def compute_chunk(pv, tv, acc):

        def group_body(g, a):
            gidx = g * (16 * _N) + idx90

            def gp(c):
                return plsc.load_gather(pv, [gidx + c])

            def gt(c):
                return plsc.load_gather(tv, [gidx + c])

            t4 = gt(4)
            obj = (t4 > 0.0).astype(jnp.float32)

            cls = zero
            for c in range(10, _N):
                d = gp(c) - gt(c)
                cls = cls + d * d

            p0, p1, p2, p3, p4 = gp(0), gp(1), gp(2), gp(3), gp(4)
            p5, p6, p7, p8, p9 = gp(5), gp(6), gp(7), gp(8), gp(9)
            t0, t1, t2, t3 = gt(0), gt(1), gt(2), gt(3)
            t5, t6, t7, t8, t9 = gt(5), gt(6), gt(7), gt(8), gt(9)

            ax1 = p0 / _S - 0.5 * p2
            ay1 = p1 / _S - 0.5 * p3
            ax2 = ax1 / _S + 0.5 * p2
            ay2 = ay1 / _S + 0.5 * p3
            bx1 = p5 / _S - 0.5 * p7
            by1 = p6 / _S - 0.5 * p8
            bx2 = bx1 / _S + 0.5 * p7
            by2 = by1 / _S + 0.5 * p8
            tx1 = t0 / _S - 0.5 * t2
            ty1 = t1 / _S - 0.5 * t3
            tx2 = tx1 / _S + 0.5 * t2
            ty2 = ty1 / _S + 0.5 * t3
            at = (tx2 - tx1) * (ty2 - ty1)

            def iou(x1, y1, x2, y2):
                wx = jnp.maximum(jnp.minimum(x2, tx2) - jnp.maximum(x1, tx1), 0.0)
                wy = jnp.maximum(jnp.minimum(y2, ty2) - jnp.maximum(y1, ty1), 0.0)
                inter = wx * wy
                ap = (x2 - x1) * (y2 - y1)
                return inter / (ap + at - inter)

            iou0 = iou(ax1, ay1, ax2, ay2)
            iou1 = iou(bx1, by1, bx2, by2)
            sel = iou1 > iou0
            max_iou = jnp.where(sel, iou1, iou0)

            dx1 = jnp.where(sel, bx1, ax1) - jnp.where(sel, t5, tx1)
            dy1 = jnp.where(sel, by1, ay1) - jnp.where(sel, t6, ty1)
            dx2 = jnp.where(sel, bx2, ax2) - jnp.where(sel, t7, tx2)
            dy2 = jnp.where(sel, by2, ay2) - jnp.where(sel, t8, ty2)
            rpc = jnp.where(sel, p9, p4)

            xywh = dx1 * dx1 + dy1 * dy1 + dx2 * dx2 + dy2 * dy2
            cc = rpc - max_iou
            d4 = p4 - t4
            d9 = p9 - t9
            conf = d4 * d4 + d9 * d9

            contrib = obj * (cls + _LC * xywh + cc * cc) + (_LN * (1.0 - obj)) * conf
            return a + contrib

        return lax.fori_loop(0, _GRP, group_body, acc)

    start_chunk(0, pv0, tv0, sem0)

    def pair_body(kk, acc):
        start_chunk(2 * kk + 1, pv1, tv1, sem1)
        wait_chunk(pv0, tv0, sem0)
        acc = compute_chunk(pv0, tv0, acc)

        @pl.when(kk < _NCH // 2 - 1)
        def _():
            start_chunk(2 * kk + 2, pv0, tv0, sem0)

        wait_chunk(pv1, tv1, sem1)
        return compute_chunk(pv1, tv1, acc)

    acc = lax.fori_loop(0, _NCH // 2, pair_body, zero)
    acc_v[...] = acc
    pltpu.sync_copy(acc_v, out_hbm.at[wid])


def kernel(pred_tensor, target_tensor):
    flat = jnp.concatenate(
        [pred_tensor.reshape(-1), target_tensor.reshape(-1)]
    )
    parts = _sc_loss(flat)
    return jnp.sum(parts) * (1.0 / _BATCH)


# speedup vs baseline: 1.2193x; 1.2193x over previous
import functools

import jax
import jax.numpy as jnp
from jax import lax
from jax.experimental import pallas as pl
from jax.experimental.pallas import tpu as pltpu
from jax.experimental.pallas import tpu_sc as plsc

_S = 7.0
_N = 90
_LC = 5.0
_LN = 0.5
_BATCH = 4096
_M = _BATCH * 49
_NW = 32
_CELLS_W = _M // _NW
_CHUNK = 224
_NCH = _CELLS_W // _CHUNK
_CW = _CHUNK * _N
_GRP = _CHUNK // 16

_mesh = plsc.VectorSubcoreMesh(core_axis_name="c", subcore_axis_name="s")


_TOFF = _M * _N


@functools.partial(
    pl.kernel,
    out_type=jax.ShapeDtypeStruct((_NW, 16), jnp.float32),
    mesh=_mesh,
    scratch_types=[
        pltpu.VMEM((_CW,), jnp.float32),
        pltpu.VMEM((_CW,), jnp.float32),
        pltpu.VMEM((_CW,), jnp.float32),
        pltpu.VMEM((_CW,), jnp.float32),
        pltpu.VMEM((16,), jnp.float32),
        pltpu.SemaphoreType.DMA,
        pltpu.SemaphoreType.DMA,
    ],
    compiler_params=pltpu.CompilerParams(needs_layout_passes=False),
)
def _sc_loss(p_hbm, t_hbm, out_hbm, pv0, tv0, pv1, tv1, acc_v, sem0, sem1):
    wid = lax.axis_index("s") * 2 + lax.axis_index("c")
    iota = lax.broadcasted_iota(jnp.int32, (16,), 0)
    idx90 = iota * _N
    zero = jnp.zeros((16,), jnp.float32)

    def start_chunk(k, pv, tv, sem):
        start = (wid * _CELLS_W + k * _CHUNK) * _N
        pltpu.async_copy(p_hbm.at[pl.ds(start, _CW)], pv, sem)
        pltpu.async_copy(t_hbm.at[pl.ds(start, _CW)], tv, sem)

    def wait_chunk(pv, tv, sem):
        pltpu.make_async_copy(p_hbm.at[pl.ds(0, _CW)], pv, sem).wait()
        pltpu.make_async_copy(t_hbm.at[pl.ds(0, _CW)], tv, sem).wait()

    def compute_chunk(pv, tv, acc):

        def group_body(g, a):
            gidx = g * (16 * _N) + idx90

            def gp(c):
                return plsc.load_gather(pv, [gidx + c])

            def gt(c):
                return plsc.load_gather(tv, [gidx + c])

            t4 = gt(4)
            obj = (t4 > 0.0).astype(jnp.float32)

            cls = zero
            for c in range(10, _N):
                d = gp(c) - gt(c)
                cls = cls + d * d

            p0, p1, p2, p3, p4 = gp(0), gp(1), gp(2), gp(3), gp(4)
            p5, p6, p7, p8, p9 = gp(5), gp(6), gp(7), gp(8), gp(9)
            t0, t1, t2, t3 = gt(0), gt(1), gt(2), gt(3)
            t5, t6, t7, t8, t9 = gt(5), gt(6), gt(7), gt(8), gt(9)

            ax1 = p0 / _S - 0.5 * p2
            ay1 = p1 / _S - 0.5 * p3
            ax2 = ax1 / _S + 0.5 * p2
            ay2 = ay1 / _S + 0.5 * p3
            bx1 = p5 / _S - 0.5 * p7
            by1 = p6 / _S - 0.5 * p8
            bx2 = bx1 / _S + 0.5 * p7
            by2 = by1 / _S + 0.5 * p8
            tx1 = t0 / _S - 0.5 * t2
            ty1 = t1 / _S - 0.5 * t3
            tx2 = tx1 / _S + 0.5 * t2
            ty2 = ty1 / _S + 0.5 * t3
            at = (tx2 - tx1) * (ty2 - ty1)

            def iou(x1, y1, x2, y2):
                wx = jnp.maximum(jnp.minimum(x2, tx2) - jnp.maximum(x1, tx1), 0.0)
                wy = jnp.maximum(jnp.minimum(y2, ty2) - jnp.maximum(y1, ty1), 0.0)
                inter = wx * wy
                ap = (x2 - x1) * (y2 - y1)
                return inter / (ap + at - inter)

            iou0 = iou(ax1, ay1, ax2, ay2)
            iou1 = iou(bx1, by1, bx2, by2)
            sel = iou1 > iou0
            max_iou = jnp.where(sel, iou1, iou0)

            dx1 = jnp.where(sel, bx1, ax1) - jnp.where(sel, t5, tx1)
            dy1 = jnp.where(sel, by1, ay1) - jnp.where(sel, t6, ty1)
            dx2 = jnp.where(sel, bx2, ax2) - jnp.where(sel, t7, tx2)
            dy2 = jnp.where(sel, by2, ay2) - jnp.where(sel, t8, ty2)
            rpc = jnp.where(sel, p9, p4)

            xywh = dx1 * dx1 + dy1 * dy1 + dx2 * dx2 + dy2 * dy2
            cc = rpc - max_iou
            d4 = p4 - t4
            d9 = p9 - t9
            conf = d4 * d4 + d9 * d9

            contrib = obj * (cls + _LC * xywh + cc * cc) + (_LN * (1.0 - obj)) * conf
            return a + contrib

        return lax.fori_loop(0, _GRP, group_body, acc)

    start_chunk(0, pv0, tv0, sem0)

    def pair_body(kk, acc):
        start_chunk(2 * kk + 1, pv1, tv1, sem1)
        wait_chunk(pv0, tv0, sem0)
        acc = compute_chunk(pv0, tv0, acc)

        @pl.when(kk < _NCH // 2 - 1)
        def _():
            start_chunk(2 * kk + 2, pv0, tv0, sem0)

        wait_chunk(pv1, tv1, sem1)
        return compute_chunk(pv1, tv1, acc)

    acc = lax.fori_loop(0, _NCH // 2, pair_body, zero)
    acc_v[...] = acc
    pltpu.sync_copy(acc_v, out_hbm.at[wid])


def kernel(pred_tensor, target_tensor):
    parts = _sc_loss(pred_tensor.reshape(-1), target_tensor.reshape(-1))
    return jnp.sum(parts) * (1.0 / _BATCH)
